# baseline (speedup 1.0000x reference)
import functools

import jax
import jax.numpy as jnp
from jax import lax
from jax.experimental import pallas as pl
from jax.experimental.pallas import tpu as pltpu
from jax.experimental.pallas import tpu_sc as plsc

N = 10000
E = 320000
H = 4
D = 32
DIM = 128

NC = 2
NS = 16
NW = NC * NS
EPW = E // NW
N4 = N * H
DN = 40960
DSTRIPE = DN // NS
NR = 10112
RSTRIPE = NR // NS

CA = 1000
CB = 184
NCHB = 54
EPI = EPW - NCHB * CB

_params = pltpu.CompilerParams(needs_layout_passes=False)


def _pre_body(x_ref, w_ref, alm_ref, arm_ref, feat_ref, el_ref, er_ref):
    f = jnp.dot(x_ref[...], w_ref[...], preferred_element_type=jnp.float32)
    feat_ref[...] = f
    el_ref[...] = jnp.dot(f, alm_ref[...], preferred_element_type=jnp.float32)
    er_ref[...] = jnp.dot(f, arm_ref[...], preferred_element_type=jnp.float32)


def _tc_pre(x, W, alm, arm):
    R = 1000
    return pl.pallas_call(
        _pre_body,
        grid=(N // R,),
        in_specs=[
            pl.BlockSpec((R, DIM), lambda i: (i, 0)),
            pl.BlockSpec((DIM, DIM), lambda i: (0, 0)),
            pl.BlockSpec((DIM, H), lambda i: (0, 0)),
            pl.BlockSpec((DIM, H), lambda i: (0, 0)),
        ],
        out_specs=[
            pl.BlockSpec((R, DIM), lambda i: (i, 0)),
            pl.BlockSpec((R, H), lambda i: (i, 0)),
            pl.BlockSpec((R, H), lambda i: (i, 0)),
        ],
        out_shape=[
            jax.ShapeDtypeStruct((N, DIM), jnp.float32),
            jax.ShapeDtypeStruct((N, H), jnp.float32),
            jax.ShapeDtypeStruct((N, H), jnp.float32),
        ],
    )(x, W, alm, arm)


def _post_body(pa_ref, pb_ref, x_ref, o_ref):
    r = pa_ref[...] + pb_ref[...] + x_ref[...]
    o_ref[...] = jnp.where(r > 0.0, r, jnp.exp(r) - 1.0)


def _tc_post(pa, pb, x):
    R = 1000
    return pl.pallas_call(
        _post_body,
        grid=(N // R,),
        in_specs=[
            pl.BlockSpec((R, DIM), lambda i: (i, 0)),
            pl.BlockSpec((R, DIM), lambda i: (i, 0)),
            pl.BlockSpec((R, DIM), lambda i: (i, 0)),
        ],
        out_specs=pl.BlockSpec((R, DIM), lambda i: (i, 0)),
        out_shape=jax.ShapeDtypeStruct((N, DIM), jnp.float32),
    )(pa, pb, x)


def _pass_a_body(src0, dst0, src1, dst1, el0, er0, el1, er1,
                 ex0, ex1, dnA0, dnB0, dnA1, dnB1,
                 src_a, dst_a, exv_a, idx4_a, src_b, dst_b, exv_b, idx4_b,
                 el_t, er_t, dn_sh,
                 lsem_a, lsem_b, stsem_a, stsem_b, scsem_a, scsem_b):
    core = lax.axis_index("c")
    sid = lax.axis_index("s")
    wid = sid * NC + core
    i16 = jnp.arange(16, dtype=jnp.int32)
    z16 = jnp.zeros((16,), jnp.float32)

    bufs_a = (src_a, dst_a, exv_a, idx4_a, lsem_a, stsem_a, scsem_a)
    bufs_b = (src_b, dst_b, exv_b, idx4_b, lsem_b, stsem_b, scsem_b)
    NCHA = EPW // CA

    def layer(src_h, dst_h, el_h, er_h, ex_h, dnA, dnB):
        c1 = pltpu.async_copy(el_h, el_t, lsem_a)
        c2 = pltpu.async_copy(er_h, er_t, lsem_b)

        def zb(j, _):
            exv_a[pl.ds(j * 16, 16)] = z16
            return 0

        lax.fori_loop(0, DSTRIPE // 16, zb, 0)
        pltpu.sync_copy(exv_a.at[pl.ds(0, DSTRIPE)],
                        dn_sh.at[pl.ds(sid * DSTRIPE, DSTRIPE)])
        c1.wait()
        c2.wait()
        plsc.subcore_barrier()

        lbase = wid * EPW

        def start_idx(t, sv, dv, sem):
            base = lbase + t * CA
            pltpu.async_copy(src_h.at[pl.ds(base, CA)], sv, sem)
            pltpu.async_copy(dst_h.at[pl.ds(base, CA)], dv, sem)

        def wait_idx(sv, dv, sem):
            pltpu.make_async_copy(src_h.at[pl.ds(0, CA)], sv, sem).wait()
            pltpu.make_async_copy(dst_h.at[pl.ds(0, CA)], dv, sem).wait()

        def step(t, cur, nxt):
            svc, dvc, exc, idc, lsc, stc, scc = cur
            svn, dvn, exn, idn, lsn, stn, scn = nxt

            @pl.when(t > 0)
            def _():
                base1 = lbase + (t - 1) * CA
                pltpu.make_async_copy(
                    exn, ex_h.at[pl.ds(base1 * 4, CA * 4)], stn).wait()
                pltpu.make_async_copy(exn, dn_sh.at[idn], scn).wait()

            @pl.when(t + 1 < NCHA)
            def _():
                start_idx(t + 1, svn, dvn, lsn)

            wait_idx(svc, dvc, lsc)

            def eb(j, _):
                p = j * 16 + i16
                k = p >> 2
                h = p & 3
                sv = plsc.load_gather(svc, [k])
                dv = plsc.load_gather(dvc, [k])
                e = (plsc.load_gather(el_t, [sv * 4 + h])
                     + plsc.load_gather(er_t, [dv * 4 + h]))
                e = jnp.where(e >= 0.0, e, 0.2 * e)
                exc[pl.ds(j * 16, 16)] = jnp.exp(e)
                idc[pl.ds(j * 16, 16)] = dv * 4 + h
                return 0

            lax.fori_loop(0, CA * H // 16, eb, 0)
            base = lbase + t * CA
            pltpu.async_copy(exc, ex_h.at[pl.ds(base * 4, CA * 4)], stc)
            pltpu.async_copy(exc, dn_sh.at[idc], scc, add=True)

        start_idx(0, src_a, dst_a, lsem_a)

        def pair(i, _):
            step(2 * i, bufs_a, bufs_b)
            step(2 * i + 1, bufs_b, bufs_a)
            return 0

        lax.fori_loop(0, NCHA // 2, pair, 0)
        base9 = lbase + (NCHA - 1) * CA
        pltpu.make_async_copy(exv_b, ex_h.at[pl.ds(base9 * 4, CA * 4)],
                              stsem_b).wait()
        pltpu.make_async_copy(exv_b, dn_sh.at[idx4_b], scsem_b).wait()
        plsc.subcore_barrier()

        @pl.when(core == 0)
        def _():
            pltpu.sync_copy(dn_sh.at[pl.ds(sid * DSTRIPE, DSTRIPE)],
                            dnA.at[pl.ds(sid * DSTRIPE, DSTRIPE)])

        @pl.when(core == 1)
        def _():
            pltpu.sync_copy(dn_sh.at[pl.ds(sid * DSTRIPE, DSTRIPE)],
                            dnB.at[pl.ds(sid * DSTRIPE, DSTRIPE)])

        plsc.subcore_barrier()

    layer(src0, dst0, el0, er0, ex0, dnA0, dnB0)
    layer(src1, dst1, el1, er1, ex1, dnA1, dnB1)


@functools.lru_cache(maxsize=None)
def _pass_a():
    mesh = plsc.VectorSubcoreMesh(core_axis_name="c", subcore_axis_name="s",
                                  num_cores=NC, num_subcores=NS)
    return pl.kernel(
        _pass_a_body,
        out_type=[
            jax.ShapeDtypeStruct((E * H,), jnp.float32),
            jax.ShapeDtypeStruct((E * H,), jnp.float32),
            jax.ShapeDtypeStruct((DN,), jnp.float32),
            jax.ShapeDtypeStruct((DN,), jnp.float32),
            jax.ShapeDtypeStruct((DN,), jnp.float32),
            jax.ShapeDtypeStruct((DN,), jnp.float32),
        ],
        mesh=mesh,
        scratch_types=[
            pltpu.VMEM((CA,), jnp.int32),
            pltpu.VMEM((CA,), jnp.int32),
            pltpu.VMEM((CA * H,), jnp.float32),
            pltpu.VMEM((CA * H,), jnp.int32),
            pltpu.VMEM((CA,), jnp.int32),
            pltpu.VMEM((CA,), jnp.int32),
            pltpu.VMEM((CA * H,), jnp.float32),
            pltpu.VMEM((CA * H,), jnp.int32),
            pltpu.VMEM((N4,), jnp.float32),
            pltpu.VMEM((N4,), jnp.float32),
            pltpu.VMEM_SHARED((DN,), jnp.float32),
            pltpu.SemaphoreType.DMA,
            pltpu.SemaphoreType.DMA,
            pltpu.SemaphoreType.DMA,
            pltpu.SemaphoreType.DMA,
            pltpu.SemaphoreType.DMA,
            pltpu.SemaphoreType.DMA,
        ],
        compiler_params=_params,
    )


def _pass_a2_body(dst0, dst1, ex0, ex1, dnA0, dnB0, dnA1, dnB1,
                  al0, al1,
                  dst_a, exv_a, av_a, dst_b, exv_b, av_b,
                  b1, b2, dn_t, lsem_a, lsem_b, stsem_a, stsem_b):
    core = lax.axis_index("c")
    sid = lax.axis_index("s")
    wid = sid * NC + core
    i16 = jnp.arange(16, dtype=jnp.int32)

    bufs_a = (dst_a, exv_a, av_a, lsem_a, stsem_a)
    bufs_b = (dst_b, exv_b, av_b, lsem_b, stsem_b)
    NCHA = EPW // CA

    def layer(dst_h, ex_h, dnA, dnB, al_h):
        def sb(q, _):
            c1 = pltpu.async_copy(dnA.at[pl.ds(q * 4000, 4000)], b1, lsem_a)
            c2 = pltpu.async_copy(dnB.at[pl.ds(q * 4000, 4000)], b2, lsem_b)
            c1.wait()
            c2.wait()

            def ib(j, _):
                dn_t[pl.ds(q * 4000 + j * 16, 16)] = (
                    b1[pl.ds(j * 16, 16)] + b2[pl.ds(j * 16, 16)])
                return 0

            lax.fori_loop(0, 250, ib, 0)
            return 0

        lax.fori_loop(0, N4 // 4000, sb, 0)

        lbase = wid * EPW

        def start_idx(t, dv, exv, sem):
            base = lbase + t * CA
            pltpu.async_copy(dst_h.at[pl.ds(base, CA)], dv, sem)
            pltpu.async_copy(ex_h.at[pl.ds(base * 4, CA * 4)], exv, sem)

        def wait_idx(dv, exv, sem):
            pltpu.make_async_copy(dst_h.at[pl.ds(0, CA)], dv, sem).wait()
            pltpu.make_async_copy(ex_h.at[pl.ds(0, CA * 4)], exv, sem).wait()

        def step(t, cur, nxt):
            dvc, exc, avc, lsc, stc = cur
            dvn, exn, avn, lsn, stn = nxt

            @pl.when(t > 0)
            def _():
                base1 = lbase + (t - 1) * CA
                pltpu.make_async_copy(
                    avn, al_h.at[pl.ds(base1 * 4, CA * 4)], stn).wait()

            @pl.when(t + 1 < NCHA)
            def _():
                start_idx(t + 1, dvn, exn, lsn)

            wait_idx(dvc, exc, lsc)

            def ab(j, _):
                p = j * 16 + i16
                k = p >> 2
                h = p & 3
                dv = plsc.load_gather(dvc, [k])
                dn = plsc.load_gather(dn_t, [dv * 4 + h])
                avc[pl.ds(j * 16, 16)] = exc[pl.ds(j * 16, 16)] / dn
                return 0

            lax.fori_loop(0, CA * H // 16, ab, 0)
            base = lbase + t * CA
            pltpu.async_copy(avc, al_h.at[pl.ds(base * 4, CA * 4)], stc)

        start_idx(0, dst_a, exv_a, lsem_a)

        def pair(i, _):
            step(2 * i, bufs_a, bufs_b)
            step(2 * i + 1, bufs_b, bufs_a)
            return 0

        lax.fori_loop(0, NCHA // 2, pair, 0)
        base9 = lbase + (NCHA - 1) * CA
        pltpu.make_async_copy(av_b, al_h.at[pl.ds(base9 * 4, CA * 4)],
                              stsem_b).wait()

    layer(dst0, ex0, dnA0, dnB0, al0)
    layer(dst1, ex1, dnA1, dnB1, al1)


@functools.lru_cache(maxsize=None)
def _pass_a2():
    mesh = plsc.VectorSubcoreMesh(core_axis_name="c", subcore_axis_name="s",
                                  num_cores=NC, num_subcores=NS)
    return pl.kernel(
        _pass_a2_body,
        out_type=[
            jax.ShapeDtypeStruct((E * H,), jnp.float32),
            jax.ShapeDtypeStruct((E * H,), jnp.float32),
        ],
        mesh=mesh,
        scratch_types=[
            pltpu.VMEM((CA,), jnp.int32),
            pltpu.VMEM((CA * H,), jnp.float32),
            pltpu.VMEM((CA * H,), jnp.float32),
            pltpu.VMEM((CA,), jnp.int32),
            pltpu.VMEM((CA * H,), jnp.float32),
            pltpu.VMEM((CA * H,), jnp.float32),
            pltpu.VMEM((4000,), jnp.float32),
            pltpu.VMEM((4000,), jnp.float32),
            pltpu.VMEM((N4,), jnp.float32),
            pltpu.SemaphoreType.DMA,
            pltpu.SemaphoreType.DMA,
            pltpu.SemaphoreType.DMA,
            pltpu.SemaphoreType.DMA,
        ],
        compiler_params=_params,
    )


def _pass_b_body(src0, dst0, src1, dst1, al0, al1, feat0, feat1,
                 rstA0, rstB0, rstA1, rstB1,
                 src_a, dst_a, av_a, fb_a, src_b, dst_b, av_b, fb_b,
                 srcE, dstE, avE, rst_sh,
                 lsem_a, lsem_b, fsem_a, fsem_b, ssem_a, ssem_b):
    core = lax.axis_index("c")
    sid = lax.axis_index("s")
    wid = sid * NC + core
    z16 = jnp.zeros((16,), jnp.float32)

    bufs_a = (src_a, dst_a, av_a, fb_a, lsem_a, fsem_a, ssem_a)
    bufs_b = (src_b, dst_b, av_b, fb_b, lsem_b, fsem_b, ssem_b)

    def layer(src_h, dst_h, al_h, feat_h, rstA, rstB):
        def zrow(k, _):
            for g in range(8):
                fb_a[k, pl.ds(g * 16, 16)] = z16
            return 0

        lax.fori_loop(0, CB, zrow, 0)
        for j in range(3):
            pltpu.sync_copy(fb_a, rst_sh.at[pl.ds(sid * RSTRIPE + j * CB, CB)])
        pltpu.sync_copy(fb_a.at[pl.ds(0, RSTRIPE - 3 * CB)],
                        rst_sh.at[pl.ds(sid * RSTRIPE + 3 * CB,
                                        RSTRIPE - 3 * CB)])
        plsc.subcore_barrier()

        lbase = wid * EPW

        def start_idx(t, sv, dv, avv, sem):
            base = lbase + t * CB
            pltpu.async_copy(src_h.at[pl.ds(base, CB)], sv, sem)
            pltpu.async_copy(dst_h.at[pl.ds(base, CB)], dv, sem)
            pltpu.async_copy(al_h.at[pl.ds(base * 4, CB * 4)], avv, sem)

        def wait_idx(sv, dv, avv, sem):
            pltpu.make_async_copy(src_h.at[pl.ds(0, CB)], sv, sem).wait()
            pltpu.make_async_copy(dst_h.at[pl.ds(0, CB)], dv, sem).wait()
            pltpu.make_async_copy(al_h.at[pl.ds(0, CB * 4)], avv, sem).wait()

        def compute(fb, avv, n):
            def eb(q, _):
                for u in range(2):
                    k = q * 2 + u
                    for h in range(H):
                        s = plsc.load_gather(avv, [jnp.full((16,), k * 4 + h,
                                                            jnp.int32)])
                        for g in range(2):
                            c0 = h * D + g * 16
                            fb[k, pl.ds(c0, 16)] = fb[k, pl.ds(c0, 16)] * s
                return 0

            lax.fori_loop(0, n // 2, eb, 0)

        def step(t, cur, nxt):
            svc, dvc, avc, fbc, lsc, fsc, ssc = cur
            svn, dvn, avn, fbn, lsn, fsn, ssn = nxt

            @pl.when(t > 0)
            def _():
                pltpu.make_async_copy(fbn, rst_sh.at[dvn], ssn).wait()

            @pl.when(t + 1 < NCHB)
            def _():
                start_idx(t + 1, svn, dvn, avn, lsn)

            pltpu.make_async_copy(feat_h.at[svc], fbc, fsc).wait()
            compute(fbc, avc, CB)

            @pl.when(t + 1 < NCHB)
            def _():
                wait_idx(svn, dvn, avn, lsn)
                pltpu.async_copy(feat_h.at[svn], fbn, fsn)

            pltpu.async_copy(fbc, rst_sh.at[dvc], ssc, add=True)

        start_idx(0, src_a, dst_a, av_a, lsem_a)
        wait_idx(src_a, dst_a, av_a, lsem_a)
        pltpu.async_copy(feat_h.at[src_a], fb_a, fsem_a)

        def pair(i, _):
            step(2 * i, bufs_a, bufs_b)
            step(2 * i + 1, bufs_b, bufs_a)
            return 0

        lax.fori_loop(0, NCHB // 2, pair, 0)
        pltpu.make_async_copy(fb_b, rst_sh.at[dst_b], ssem_b).wait()

        ebase = lbase + NCHB * CB
        pltpu.sync_copy(src_h.at[pl.ds(ebase, EPI)], srcE)
        pltpu.sync_copy(dst_h.at[pl.ds(ebase, EPI)], dstE)
        pltpu.sync_copy(al_h.at[pl.ds(ebase * 4, EPI * 4)], avE)
        pltpu.async_copy(feat_h.at[srcE], fb_a.at[pl.ds(0, EPI)],
                         fsem_a).wait()
        compute(fb_a, avE, EPI)
        pltpu.sync_copy(fb_a.at[pl.ds(0, EPI)], rst_sh.at[dstE], add=True)

        plsc.subcore_barrier()

        @pl.when(core == 0)
        def _():
            pltpu.sync_copy(rst_sh.at[pl.ds(sid * RSTRIPE, RSTRIPE)],
                            rstA.at[pl.ds(sid * RSTRIPE, RSTRIPE)])

        @pl.when(core == 1)
        def _():
            pltpu.sync_copy(rst_sh.at[pl.ds(sid * RSTRIPE, RSTRIPE)],
                            rstB.at[pl.ds(sid * RSTRIPE, RSTRIPE)])

        plsc.subcore_barrier()

    layer(src0, dst0, al0, feat0, rstA0, rstB0)
    layer(src1, dst1, al1, feat1, rstA1, rstB1)


@functools.lru_cache(maxsize=None)
def _pass_b():
    mesh = plsc.VectorSubcoreMesh(core_axis_name="c", subcore_axis_name="s",
                                  num_cores=NC, num_subcores=NS)
    return pl.kernel(
        _pass_b_body,
        out_type=[
            jax.ShapeDtypeStruct((NR, DIM), jnp.float32),
            jax.ShapeDtypeStruct((NR, DIM), jnp.float32),
            jax.ShapeDtypeStruct((NR, DIM), jnp.float32),
            jax.ShapeDtypeStruct((NR, DIM), jnp.float32),
        ],
        mesh=mesh,
        scratch_types=[
            pltpu.VMEM((CB,), jnp.int32),
            pltpu.VMEM((CB,), jnp.int32),
            pltpu.VMEM((CB * H,), jnp.float32),
            pltpu.VMEM((CB, DIM), jnp.float32),
            pltpu.VMEM((CB,), jnp.int32),
            pltpu.VMEM((CB,), jnp.int32),
            pltpu.VMEM((CB * H,), jnp.float32),
            pltpu.VMEM((CB, DIM), jnp.float32),
            pltpu.VMEM((EPI,), jnp.int32),
            pltpu.VMEM((EPI,), jnp.int32),
            pltpu.VMEM((EPI * H,), jnp.float32),
            pltpu.VMEM_SHARED((NR, DIM), jnp.float32),
            pltpu.SemaphoreType.DMA,
            pltpu.SemaphoreType.DMA,
            pltpu.SemaphoreType.DMA,
            pltpu.SemaphoreType.DMA,
            pltpu.SemaphoreType.DMA,
            pltpu.SemaphoreType.DMA,
        ],
        compiler_params=_params,
    )


def _expand_att(a):
    rows = jnp.arange(DIM)
    m = (rows[:, None] // D) == jnp.arange(H)[None, :]
    return jnp.where(m, a.reshape(-1)[:, None], 0.0).astype(jnp.float32)


def kernel(x0, x1, edge_index0, edge_index1, W0, al0, ar0, W1, al1, ar1):
    feat0, el0, er0 = _tc_pre(x0, W0, _expand_att(al0), _expand_att(ar0))
    feat1, el1, er1 = _tc_pre(x1, W1, _expand_att(al1), _expand_att(ar1))

    src0, dst0 = edge_index0[0], edge_index0[1]
    src1, dst1 = edge_index1[0], edge_index1[1]

    ex0, ex1, dnA0, dnB0, dnA1, dnB1 = _pass_a()(
        src0, dst0, src1, dst1,
        el0.reshape(-1), er0.reshape(-1), el1.reshape(-1), er1.reshape(-1))

    al0_, al1_ = _pass_a2()(
        dst0, dst1, ex0, ex1, dnA0, dnB0, dnA1, dnB1)

    rstA0, rstB0, rstA1, rstB1 = _pass_b()(
        src0, dst0, src1, dst1, al0_, al1_, feat0, feat1)

    h0 = _tc_post(rstA0, rstB0, x0)
    h1 = _tc_post(rstA1, rstB1, x1)

    return (h0, h1,
            al0_.reshape(E, H, 1), al1_.reshape(E, H, 1))


# speedup vs baseline: 1.1041x; 1.1041x over previous
import functools

import jax
import jax.numpy as jnp
from jax import lax
from jax.experimental import pallas as pl
from jax.experimental.pallas import tpu as pltpu
from jax.experimental.pallas import tpu_sc as plsc

N = 10000
E = 320000
H = 4
D = 32
DIM = 128

NC = 2
NS = 16
NW = NC * NS
EPW = E // NW
N4 = N * H
DN = 40960
DSTRIPE = DN // NS
NR = 10112
RSTRIPE = NR // NS

CA = 1000
CB = 184
NCHB = 54
EPI = EPW - NCHB * CB

_params = pltpu.CompilerParams(needs_layout_passes=False)


def _pre_body(x_ref, w_ref, alm_ref, arm_ref, feat_ref, el_ref, er_ref):
    f = jnp.dot(x_ref[...], w_ref[...], preferred_element_type=jnp.float32)
    feat_ref[...] = f
    el_ref[...] = jnp.dot(f, alm_ref[...], preferred_element_type=jnp.float32)
    er_ref[...] = jnp.dot(f, arm_ref[...], preferred_element_type=jnp.float32)


def _tc_pre(x, W, alm, arm):
    R = 1000
    return pl.pallas_call(
        _pre_body,
        grid=(N // R,),
        in_specs=[
            pl.BlockSpec((R, DIM), lambda i: (i, 0)),
            pl.BlockSpec((DIM, DIM), lambda i: (0, 0)),
            pl.BlockSpec((DIM, H), lambda i: (0, 0)),
            pl.BlockSpec((DIM, H), lambda i: (0, 0)),
        ],
        out_specs=[
            pl.BlockSpec((R, DIM), lambda i: (i, 0)),
            pl.BlockSpec((R, H), lambda i: (i, 0)),
            pl.BlockSpec((R, H), lambda i: (i, 0)),
        ],
        out_shape=[
            jax.ShapeDtypeStruct((N, DIM), jnp.float32),
            jax.ShapeDtypeStruct((N, H), jnp.float32),
            jax.ShapeDtypeStruct((N, H), jnp.float32),
        ],
    )(x, W, alm, arm)


def _post_body(pa_ref, pb_ref, x_ref, o_ref):
    r = pa_ref[...] + pb_ref[...] + x_ref[...]
    o_ref[...] = jnp.where(r > 0.0, r, jnp.exp(r) - 1.0)


def _tc_post(pa, pb, x):
    R = 1000
    return pl.pallas_call(
        _post_body,
        grid=(N // R,),
        in_specs=[
            pl.BlockSpec((R, DIM), lambda i: (i, 0)),
            pl.BlockSpec((R, DIM), lambda i: (i, 0)),
            pl.BlockSpec((R, DIM), lambda i: (i, 0)),
        ],
        out_specs=pl.BlockSpec((R, DIM), lambda i: (i, 0)),
        out_shape=jax.ShapeDtypeStruct((N, DIM), jnp.float32),
    )(pa, pb, x)


def _pass_a_body(src0, dst0, src1, dst1, el0, er0, el1, er1,
                 ex0, ex1, dnA0, dnB0, dnA1, dnB1,
                 src_a, dst_a, exv_a, idx4_a, src_b, dst_b, exv_b, idx4_b,
                 el_t, er_t, dn_sh,
                 lsem_a, lsem_b, stsem_a, stsem_b, scsem_a, scsem_b):
    core = lax.axis_index("c")
    sid = lax.axis_index("s")
    wid = sid * NC + core
    i16 = jnp.arange(16, dtype=jnp.int32)
    z16 = jnp.zeros((16,), jnp.float32)

    bufs_a = (src_a, dst_a, exv_a, idx4_a, lsem_a, stsem_a, scsem_a)
    bufs_b = (src_b, dst_b, exv_b, idx4_b, lsem_b, stsem_b, scsem_b)
    NCHA = EPW // CA

    def layer(src_h, dst_h, el_h, er_h, ex_h, dnA, dnB):
        c1 = pltpu.async_copy(el_h, el_t, lsem_a)
        c2 = pltpu.async_copy(er_h, er_t, lsem_b)

        def zb(j, _):
            exv_a[pl.ds(j * 16, 16)] = z16
            return 0

        lax.fori_loop(0, DSTRIPE // 16, zb, 0)
        pltpu.sync_copy(exv_a.at[pl.ds(0, DSTRIPE)],
                        dn_sh.at[pl.ds(sid * DSTRIPE, DSTRIPE)])
        c1.wait()
        c2.wait()
        plsc.subcore_barrier()

        lbase = wid * EPW

        def start_idx(t, sv, dv, sem):
            base = lbase + t * CA
            pltpu.async_copy(src_h.at[pl.ds(base, CA)], sv, sem)
            pltpu.async_copy(dst_h.at[pl.ds(base, CA)], dv, sem)

        def wait_idx(sv, dv, sem):
            pltpu.make_async_copy(src_h.at[pl.ds(0, CA)], sv, sem).wait()
            pltpu.make_async_copy(dst_h.at[pl.ds(0, CA)], dv, sem).wait()

        def step(t, cur, nxt):
            svc, dvc, exc, idc, lsc, stc, scc = cur
            svn, dvn, exn, idn, lsn, stn, scn = nxt

            @pl.when(t > 0)
            def _():
                base1 = lbase + (t - 1) * CA
                pltpu.make_async_copy(
                    exn, ex_h.at[pl.ds(base1 * 4, CA * 4)], stn).wait()
                pltpu.make_async_copy(exn, dn_sh.at[idn], scn).wait()

            @pl.when(t + 1 < NCHA)
            def _():
                start_idx(t + 1, svn, dvn, lsn)

            wait_idx(svc, dvc, lsc)

            def eb(j, _):
                p = j * 16 + i16
                k = p >> 2
                h = p & 3
                sv = plsc.load_gather(svc, [k])
                dv = plsc.load_gather(dvc, [k])
                e = (plsc.load_gather(el_t, [sv * 4 + h])
                     + plsc.load_gather(er_t, [dv * 4 + h]))
                e = jnp.where(e >= 0.0, e, 0.2 * e)
                exc[pl.ds(j * 16, 16)] = jnp.exp(e)
                idc[pl.ds(j * 16, 16)] = dv * 4 + h
                return 0

            lax.fori_loop(0, CA * H // 16, eb, 0)
            base = lbase + t * CA
            pltpu.async_copy(exc, ex_h.at[pl.ds(base * 4, CA * 4)], stc)
            pltpu.async_copy(exc, dn_sh.at[idc], scc, add=True)

        start_idx(0, src_a, dst_a, lsem_a)

        def pair(i, _):
            step(2 * i, bufs_a, bufs_b)
            step(2 * i + 1, bufs_b, bufs_a)
            return 0

        lax.fori_loop(0, NCHA // 2, pair, 0)
        base9 = lbase + (NCHA - 1) * CA
        pltpu.make_async_copy(exv_b, ex_h.at[pl.ds(base9 * 4, CA * 4)],
                              stsem_b).wait()
        pltpu.make_async_copy(exv_b, dn_sh.at[idx4_b], scsem_b).wait()
        plsc.subcore_barrier()

        @pl.when(core == 0)
        def _():
            pltpu.sync_copy(dn_sh.at[pl.ds(sid * DSTRIPE, DSTRIPE)],
                            dnA.at[pl.ds(sid * DSTRIPE, DSTRIPE)])

        @pl.when(core == 1)
        def _():
            pltpu.sync_copy(dn_sh.at[pl.ds(sid * DSTRIPE, DSTRIPE)],
                            dnB.at[pl.ds(sid * DSTRIPE, DSTRIPE)])

        plsc.subcore_barrier()

    layer(src0, dst0, el0, er0, ex0, dnA0, dnB0)
    layer(src1, dst1, el1, er1, ex1, dnA1, dnB1)


@functools.lru_cache(maxsize=None)
def _pass_a():
    mesh = plsc.VectorSubcoreMesh(core_axis_name="c", subcore_axis_name="s",
                                  num_cores=NC, num_subcores=NS)
    return pl.kernel(
        _pass_a_body,
        out_type=[
            jax.ShapeDtypeStruct((E * H,), jnp.float32),
            jax.ShapeDtypeStruct((E * H,), jnp.float32),
            jax.ShapeDtypeStruct((DN,), jnp.float32),
            jax.ShapeDtypeStruct((DN,), jnp.float32),
            jax.ShapeDtypeStruct((DN,), jnp.float32),
            jax.ShapeDtypeStruct((DN,), jnp.float32),
        ],
        mesh=mesh,
        scratch_types=[
            pltpu.VMEM((CA,), jnp.int32),
            pltpu.VMEM((CA,), jnp.int32),
            pltpu.VMEM((CA * H,), jnp.float32),
            pltpu.VMEM((CA * H,), jnp.int32),
            pltpu.VMEM((CA,), jnp.int32),
            pltpu.VMEM((CA,), jnp.int32),
            pltpu.VMEM((CA * H,), jnp.float32),
            pltpu.VMEM((CA * H,), jnp.int32),
            pltpu.VMEM((N4,), jnp.float32),
            pltpu.VMEM((N4,), jnp.float32),
            pltpu.VMEM_SHARED((DN,), jnp.float32),
            pltpu.SemaphoreType.DMA,
            pltpu.SemaphoreType.DMA,
            pltpu.SemaphoreType.DMA,
            pltpu.SemaphoreType.DMA,
            pltpu.SemaphoreType.DMA,
            pltpu.SemaphoreType.DMA,
        ],
        compiler_params=_params,
    )


def _pass_a2_body(dst0, dst1, ex0, ex1, dnA0, dnB0, dnA1, dnB1,
                  al0, al1,
                  dst_a, exv_a, av_a, dst_b, exv_b, av_b,
                  b1, b2, dn_t, lsem_a, lsem_b, stsem_a, stsem_b):
    core = lax.axis_index("c")
    sid = lax.axis_index("s")
    wid = sid * NC + core
    i16 = jnp.arange(16, dtype=jnp.int32)

    bufs_a = (dst_a, exv_a, av_a, lsem_a, stsem_a)
    bufs_b = (dst_b, exv_b, av_b, lsem_b, stsem_b)
    NCHA = EPW // CA

    def layer(dst_h, ex_h, dnA, dnB, al_h):
        def sb(q, _):
            c1 = pltpu.async_copy(dnA.at[pl.ds(q * 4000, 4000)], b1, lsem_a)
            c2 = pltpu.async_copy(dnB.at[pl.ds(q * 4000, 4000)], b2, lsem_b)
            c1.wait()
            c2.wait()

            def ib(j, _):
                dn_t[pl.ds(q * 4000 + j * 16, 16)] = 1.0 / (
                    b1[pl.ds(j * 16, 16)] + b2[pl.ds(j * 16, 16)])
                return 0

            lax.fori_loop(0, 250, ib, 0)
            return 0

        lax.fori_loop(0, N4 // 4000, sb, 0)

        lbase = wid * EPW

        def start_idx(t, dv, exv, sem):
            base = lbase + t * CA
            pltpu.async_copy(dst_h.at[pl.ds(base, CA)], dv, sem)
            pltpu.async_copy(ex_h.at[pl.ds(base * 4, CA * 4)], exv, sem)

        def wait_idx(dv, exv, sem):
            pltpu.make_async_copy(dst_h.at[pl.ds(0, CA)], dv, sem).wait()
            pltpu.make_async_copy(ex_h.at[pl.ds(0, CA * 4)], exv, sem).wait()

        def step(t, cur, nxt):
            dvc, exc, avc, lsc, stc = cur
            dvn, exn, avn, lsn, stn = nxt

            @pl.when(t > 0)
            def _():
                base1 = lbase + (t - 1) * CA
                pltpu.make_async_copy(
                    avn, al_h.at[pl.ds(base1 * 4, CA * 4)], stn).wait()

            @pl.when(t + 1 < NCHA)
            def _():
                start_idx(t + 1, dvn, exn, lsn)

            wait_idx(dvc, exc, lsc)

            def ab(j, _):
                p = j * 16 + i16
                k = p >> 2
                h = p & 3
                dv = plsc.load_gather(dvc, [k])
                dn = plsc.load_gather(dn_t, [dv * 4 + h])
                avc[pl.ds(j * 16, 16)] = exc[pl.ds(j * 16, 16)] * dn
                return 0

            lax.fori_loop(0, CA * H // 16, ab, 0)
            base = lbase + t * CA
            pltpu.async_copy(avc, al_h.at[pl.ds(base * 4, CA * 4)], stc)

        start_idx(0, dst_a, exv_a, lsem_a)

        def pair(i, _):
            step(2 * i, bufs_a, bufs_b)
            step(2 * i + 1, bufs_b, bufs_a)
            return 0

        lax.fori_loop(0, NCHA // 2, pair, 0)
        base9 = lbase + (NCHA - 1) * CA
        pltpu.make_async_copy(av_b, al_h.at[pl.ds(base9 * 4, CA * 4)],
                              stsem_b).wait()

    layer(dst0, ex0, dnA0, dnB0, al0)
    layer(dst1, ex1, dnA1, dnB1, al1)


@functools.lru_cache(maxsize=None)
def _pass_a2():
    mesh = plsc.VectorSubcoreMesh(core_axis_name="c", subcore_axis_name="s",
                                  num_cores=NC, num_subcores=NS)
    return pl.kernel(
        _pass_a2_body,
        out_type=[
            jax.ShapeDtypeStruct((E * H,), jnp.float32),
            jax.ShapeDtypeStruct((E * H,), jnp.float32),
        ],
        mesh=mesh,
        scratch_types=[
            pltpu.VMEM((CA,), jnp.int32),
            pltpu.VMEM((CA * H,), jnp.float32),
            pltpu.VMEM((CA * H,), jnp.float32),
            pltpu.VMEM((CA,), jnp.int32),
            pltpu.VMEM((CA * H,), jnp.float32),
            pltpu.VMEM((CA * H,), jnp.float32),
            pltpu.VMEM((4000,), jnp.float32),
            pltpu.VMEM((4000,), jnp.float32),
            pltpu.VMEM((N4,), jnp.float32),
            pltpu.SemaphoreType.DMA,
            pltpu.SemaphoreType.DMA,
            pltpu.SemaphoreType.DMA,
            pltpu.SemaphoreType.DMA,
        ],
        compiler_params=_params,
    )


def _pass_b_body(src0, dst0, src1, dst1, al0, al1, feat0, feat1,
                 rstA0, rstB0, rstA1, rstB1,
                 src_a, dst_a, av_a, fb_a, src_b, dst_b, av_b, fb_b,
                 srcE, dstE, avE, rst_sh,
                 lsem_a, lsem_b, fsem_a, fsem_b, ssem_a, ssem_b,
                 rsem_a, rsem_b):
    core = lax.axis_index("c")
    sid = lax.axis_index("s")
    wid = sid * NC + core
    z16 = jnp.zeros((16,), jnp.float32)

    bufs_a = (src_a, dst_a, av_a, fb_a, lsem_a, fsem_a, ssem_a, rsem_a)
    bufs_b = (src_b, dst_b, av_b, fb_b, lsem_b, fsem_b, ssem_b, rsem_b)

    def layer(src_h, dst_h, al_h, feat_h, rstA, rstB):
        def zrow(k, _):
            for g in range(8):
                fb_a[k, pl.ds(g * 16, 16)] = z16
            return 0

        lax.fori_loop(0, CB, zrow, 0)
        for j in range(3):
            pltpu.sync_copy(fb_a, rst_sh.at[pl.ds(sid * RSTRIPE + j * CB, CB)])
        pltpu.sync_copy(fb_a.at[pl.ds(0, RSTRIPE - 3 * CB)],
                        rst_sh.at[pl.ds(sid * RSTRIPE + 3 * CB,
                                        RSTRIPE - 3 * CB)])
        plsc.subcore_barrier()

        lbase = wid * EPW

        def start_idx(t, sv, dv, avv, sem):
            base = lbase + t * CB
            pltpu.async_copy(src_h.at[pl.ds(base, CB)], sv, sem)
            pltpu.async_copy(dst_h.at[pl.ds(base, CB)], dv, sem)
            pltpu.async_copy(al_h.at[pl.ds(base * 4, CB * 4)], avv, sem)

        def wait_idx(sv, dv, avv, sem):
            pltpu.make_async_copy(src_h.at[pl.ds(0, CB)], sv, sem).wait()
            pltpu.make_async_copy(dst_h.at[pl.ds(0, CB)], dv, sem).wait()
            pltpu.make_async_copy(al_h.at[pl.ds(0, CB * 4)], avv, sem).wait()

        def compute(fb, avv, n):
            def eb(q, _):
                for u in range(2):
                    k = q * 2 + u
                    for h in range(H):
                        s = plsc.load_gather(avv, [jnp.full((16,), k * 4 + h,
                                                            jnp.int32)])
                        for g in range(2):
                            c0 = h * D + g * 16
                            fb[k, pl.ds(c0, 16)] = fb[k, pl.ds(c0, 16)] * s
                return 0

            lax.fori_loop(0, n // 2, eb, 0)

        def step(t, cur, nxt):
            svc, dvc, avc, fbc, lsc, fsc, ssc, rsc = cur
            svn, dvn, avn, fbn, lsn, fsn, ssn, rsn = nxt

            @pl.when(t > 0)
            def _():
                pltpu.make_async_copy(fbn, rst_sh.at[dvn], ssn).wait()

            @pl.when(t + 1 < NCHB)
            def _():
                pltpu.make_async_copy(src_h.at[pl.ds(0, CB)], svn, rsn).wait()
                pltpu.async_copy(feat_h.at[svn], fbn, fsn)
                base1 = lbase + (t + 1) * CB
                pltpu.async_copy(dst_h.at[pl.ds(base1, CB)], dvn, lsn)
                pltpu.async_copy(al_h.at[pl.ds(base1 * 4, CB * 4)], avn, lsn)

            pltpu.make_async_copy(feat_h.at[svc], fbc, fsc).wait()

            @pl.when(t + 2 < NCHB)
            def _():
                base2 = lbase + (t + 2) * CB
                pltpu.async_copy(src_h.at[pl.ds(base2, CB)], svc, rsc)

            pltpu.make_async_copy(dst_h.at[pl.ds(0, CB)], dvc, lsc).wait()
            pltpu.make_async_copy(al_h.at[pl.ds(0, CB * 4)], avc, lsc).wait()
            compute(fbc, avc, CB)
            pltpu.async_copy(fbc, rst_sh.at[dvc], ssc, add=True)

        pltpu.async_copy(src_h.at[pl.ds(lbase, CB)], src_a, rsem_a)
        pltpu.async_copy(dst_h.at[pl.ds(lbase, CB)], dst_a, lsem_a)
        pltpu.async_copy(al_h.at[pl.ds(lbase * 4, CB * 4)], av_a, lsem_a)
        pltpu.make_async_copy(src_h.at[pl.ds(0, CB)], src_a, rsem_a).wait()
        pltpu.async_copy(feat_h.at[src_a], fb_a, fsem_a)
        pltpu.async_copy(src_h.at[pl.ds(lbase + CB, CB)], src_b, rsem_b)

        def pair(i, _):
            step(2 * i, bufs_a, bufs_b)
            step(2 * i + 1, bufs_b, bufs_a)
            return 0

        lax.fori_loop(0, NCHB // 2, pair, 0)
        pltpu.make_async_copy(fb_b, rst_sh.at[dst_b], ssem_b).wait()

        ebase = lbase + NCHB * CB
        pltpu.sync_copy(src_h.at[pl.ds(ebase, EPI)], srcE)
        pltpu.sync_copy(dst_h.at[pl.ds(ebase, EPI)], dstE)
        pltpu.sync_copy(al_h.at[pl.ds(ebase * 4, EPI * 4)], avE)
        pltpu.async_copy(feat_h.at[srcE], fb_a.at[pl.ds(0, EPI)],
                         fsem_a).wait()
        compute(fb_a, avE, EPI)
        pltpu.sync_copy(fb_a.at[pl.ds(0, EPI)], rst_sh.at[dstE], add=True)

        plsc.subcore_barrier()

        @pl.when(core == 0)
        def _():
            pltpu.sync_copy(rst_sh.at[pl.ds(sid * RSTRIPE, RSTRIPE)],
                            rstA.at[pl.ds(sid * RSTRIPE, RSTRIPE)])

        @pl.when(core == 1)
        def _():
            pltpu.sync_copy(rst_sh.at[pl.ds(sid * RSTRIPE, RSTRIPE)],
                            rstB.at[pl.ds(sid * RSTRIPE, RSTRIPE)])

        plsc.subcore_barrier()

    layer(src0, dst0, al0, feat0, rstA0, rstB0)
    layer(src1, dst1, al1, feat1, rstA1, rstB1)


@functools.lru_cache(maxsize=None)
def _pass_b():
    mesh = plsc.VectorSubcoreMesh(core_axis_name="c", subcore_axis_name="s",
                                  num_cores=NC, num_subcores=NS)
    return pl.kernel(
        _pass_b_body,
        out_type=[
            jax.ShapeDtypeStruct((NR, DIM), jnp.float32),
            jax.ShapeDtypeStruct((NR, DIM), jnp.float32),
            jax.ShapeDtypeStruct((NR, DIM), jnp.float32),
            jax.ShapeDtypeStruct((NR, DIM), jnp.float32),
        ],
        mesh=mesh,
        scratch_types=[
            pltpu.VMEM((CB,), jnp.int32),
            pltpu.VMEM((CB,), jnp.int32),
            pltpu.VMEM((CB * H,), jnp.float32),
            pltpu.VMEM((CB, DIM), jnp.float32),
            pltpu.VMEM((CB,), jnp.int32),
            pltpu.VMEM((CB,), jnp.int32),
            pltpu.VMEM((CB * H,), jnp.float32),
            pltpu.VMEM((CB, DIM), jnp.float32),
            pltpu.VMEM((EPI,), jnp.int32),
            pltpu.VMEM((EPI,), jnp.int32),
            pltpu.VMEM((EPI * H,), jnp.float32),
            pltpu.VMEM_SHARED((NR, DIM), jnp.float32),
            pltpu.SemaphoreType.DMA,
            pltpu.SemaphoreType.DMA,
            pltpu.SemaphoreType.DMA,
            pltpu.SemaphoreType.DMA,
            pltpu.SemaphoreType.DMA,
            pltpu.SemaphoreType.DMA,
            pltpu.SemaphoreType.DMA,
            pltpu.SemaphoreType.DMA,
        ],
        compiler_params=_params,
    )


def _expand_att(a):
    rows = jnp.arange(DIM)
    m = (rows[:, None] // D) == jnp.arange(H)[None, :]
    return jnp.where(m, a.reshape(-1)[:, None], 0.0).astype(jnp.float32)


def kernel(x0, x1, edge_index0, edge_index1, W0, al0, ar0, W1, al1, ar1):
    feat0, el0, er0 = _tc_pre(x0, W0, _expand_att(al0), _expand_att(ar0))
    feat1, el1, er1 = _tc_pre(x1, W1, _expand_att(al1), _expand_att(ar1))

    src0, dst0 = edge_index0[0], edge_index0[1]
    src1, dst1 = edge_index1[0], edge_index1[1]

    ex0, ex1, dnA0, dnB0, dnA1, dnB1 = _pass_a()(
        src0, dst0, src1, dst1,
        el0.reshape(-1), er0.reshape(-1), el1.reshape(-1), er1.reshape(-1))

    al0_, al1_ = _pass_a2()(
        dst0, dst1, ex0, ex1, dnA0, dnB0, dnA1, dnB1)

    rstA0, rstB0, rstA1, rstB1 = _pass_b()(
        src0, dst0, src1, dst1, al0_, al1_, feat0, feat1)

    h0 = _tc_post(rstA0, rstB0, x0)
    h1 = _tc_post(rstA1, rstB1, x1)

    return (h0, h1,
            al0_.reshape(E, H, 1), al1_.reshape(E, H, 1))


# speedup vs baseline: 1.1856x; 1.0738x over previous
import functools

import jax
import jax.numpy as jnp
from jax import lax
from jax.experimental import pallas as pl
from jax.experimental.pallas import tpu as pltpu
from jax.experimental.pallas import tpu_sc as plsc

N = 10000
E = 320000
H = 4
D = 32
DIM = 128

NC = 2
NS = 16
NW = NC * NS
EPW = E // NW
N4 = N * H
DN = 40960
DSTRIPE = DN // NS
NR = 10112
RSTRIPE = NR // NS

CA = 1000
CB = 104
NCHB = 96
EPI = EPW - NCHB * CB

_params = pltpu.CompilerParams(needs_layout_passes=False)


def _pre_body(x_ref, w_ref, alm_ref, arm_ref, feat_ref, el_ref, er_ref):
    f = jnp.dot(x_ref[...], w_ref[...], preferred_element_type=jnp.float32)
    feat_ref[...] = f
    el_ref[...] = jnp.dot(f, alm_ref[...], preferred_element_type=jnp.float32)
    er_ref[...] = jnp.dot(f, arm_ref[...], preferred_element_type=jnp.float32)


def _tc_pre(x, W, alm, arm):
    R = 1000
    return pl.pallas_call(
        _pre_body,
        grid=(N // R,),
        in_specs=[
            pl.BlockSpec((R, DIM), lambda i: (i, 0)),
            pl.BlockSpec((DIM, DIM), lambda i: (0, 0)),
            pl.BlockSpec((DIM, H), lambda i: (0, 0)),
            pl.BlockSpec((DIM, H), lambda i: (0, 0)),
        ],
        out_specs=[
            pl.BlockSpec((R, DIM), lambda i: (i, 0)),
            pl.BlockSpec((R, H), lambda i: (i, 0)),
            pl.BlockSpec((R, H), lambda i: (i, 0)),
        ],
        out_shape=[
            jax.ShapeDtypeStruct((N, DIM), jnp.float32),
            jax.ShapeDtypeStruct((N, H), jnp.float32),
            jax.ShapeDtypeStruct((N, H), jnp.float32),
        ],
    )(x, W, alm, arm)


def _post_body(pa_ref, pb_ref, x_ref, o_ref):
    r = pa_ref[...] + pb_ref[...] + x_ref[...]
    o_ref[...] = jnp.where(r > 0.0, r, jnp.exp(r) - 1.0)


def _tc_post(pa, pb, x):
    R = 1000
    return pl.pallas_call(
        _post_body,
        grid=(N // R,),
        in_specs=[
            pl.BlockSpec((R, DIM), lambda i: (i, 0)),
            pl.BlockSpec((R, DIM), lambda i: (i, 0)),
            pl.BlockSpec((R, DIM), lambda i: (i, 0)),
        ],
        out_specs=pl.BlockSpec((R, DIM), lambda i: (i, 0)),
        out_shape=jax.ShapeDtypeStruct((N, DIM), jnp.float32),
    )(pa, pb, x)


def _pass_a_body(src0, dst0, src1, dst1, el0, er0, el1, er1,
                 ex0, ex1, dnA0, dnB0, dnA1, dnB1,
                 src_a, dst_a, exv_a, idx4_a, src_b, dst_b, exv_b, idx4_b,
                 el_t, er_t, dn_sh,
                 lsem_a, lsem_b, stsem_a, stsem_b, scsem_a, scsem_b):
    core = lax.axis_index("c")
    sid = lax.axis_index("s")
    wid = sid * NC + core
    i16 = jnp.arange(16, dtype=jnp.int32)
    z16 = jnp.zeros((16,), jnp.float32)

    bufs_a = (src_a, dst_a, exv_a, idx4_a, lsem_a, stsem_a, scsem_a)
    bufs_b = (src_b, dst_b, exv_b, idx4_b, lsem_b, stsem_b, scsem_b)
    NCHA = EPW // CA

    def layer(src_h, dst_h, el_h, er_h, ex_h, dnA, dnB):
        c1 = pltpu.async_copy(el_h, el_t, lsem_a)
        c2 = pltpu.async_copy(er_h, er_t, lsem_b)

        def zb(j, _):
            exv_a[pl.ds(j * 16, 16)] = z16
            return 0

        lax.fori_loop(0, DSTRIPE // 16, zb, 0)
        pltpu.sync_copy(exv_a.at[pl.ds(0, DSTRIPE)],
                        dn_sh.at[pl.ds(sid * DSTRIPE, DSTRIPE)])
        c1.wait()
        c2.wait()
        plsc.subcore_barrier()

        lbase = wid * EPW

        def start_idx(t, sv, dv, sem):
            base = lbase + t * CA
            pltpu.async_copy(src_h.at[pl.ds(base, CA)], sv, sem)
            pltpu.async_copy(dst_h.at[pl.ds(base, CA)], dv, sem)

        def wait_idx(sv, dv, sem):
            pltpu.make_async_copy(src_h.at[pl.ds(0, CA)], sv, sem).wait()
            pltpu.make_async_copy(dst_h.at[pl.ds(0, CA)], dv, sem).wait()

        def step(t, cur, nxt):
            svc, dvc, exc, idc, lsc, stc, scc = cur
            svn, dvn, exn, idn, lsn, stn, scn = nxt

            @pl.when(t > 0)
            def _():
                base1 = lbase + (t - 1) * CA
                pltpu.make_async_copy(
                    exn, ex_h.at[pl.ds(base1 * 4, CA * 4)], stn).wait()
                pltpu.make_async_copy(exn, dn_sh.at[idn], scn).wait()

            @pl.when(t + 1 < NCHA)
            def _():
                start_idx(t + 1, svn, dvn, lsn)

            wait_idx(svc, dvc, lsc)

            def eb(j, _):
                p = j * 16 + i16
                k = p >> 2
                h = p & 3
                sv = plsc.load_gather(svc, [k])
                dv = plsc.load_gather(dvc, [k])
                e = (plsc.load_gather(el_t, [sv * 4 + h])
                     + plsc.load_gather(er_t, [dv * 4 + h]))
                e = jnp.where(e >= 0.0, e, 0.2 * e)
                exc[pl.ds(j * 16, 16)] = jnp.exp(e)
                idc[pl.ds(j * 16, 16)] = dv * 4 + h
                return 0

            lax.fori_loop(0, CA * H // 16, eb, 0)
            base = lbase + t * CA
            pltpu.async_copy(exc, ex_h.at[pl.ds(base * 4, CA * 4)], stc)
            pltpu.async_copy(exc, dn_sh.at[idc], scc, add=True)

        start_idx(0, src_a, dst_a, lsem_a)

        def pair(i, _):
            step(2 * i, bufs_a, bufs_b)
            step(2 * i + 1, bufs_b, bufs_a)
            return 0

        lax.fori_loop(0, NCHA // 2, pair, 0)
        base9 = lbase + (NCHA - 1) * CA
        pltpu.make_async_copy(exv_b, ex_h.at[pl.ds(base9 * 4, CA * 4)],
                              stsem_b).wait()
        pltpu.make_async_copy(exv_b, dn_sh.at[idx4_b], scsem_b).wait()
        plsc.subcore_barrier()

        @pl.when(core == 0)
        def _():
            pltpu.sync_copy(dn_sh.at[pl.ds(sid * DSTRIPE, DSTRIPE)],
                            dnA.at[pl.ds(sid * DSTRIPE, DSTRIPE)])

        @pl.when(core == 1)
        def _():
            pltpu.sync_copy(dn_sh.at[pl.ds(sid * DSTRIPE, DSTRIPE)],
                            dnB.at[pl.ds(sid * DSTRIPE, DSTRIPE)])

        plsc.subcore_barrier()

    layer(src0, dst0, el0, er0, ex0, dnA0, dnB0)
    layer(src1, dst1, el1, er1, ex1, dnA1, dnB1)


@functools.lru_cache(maxsize=None)
def _pass_a():
    mesh = plsc.VectorSubcoreMesh(core_axis_name="c", subcore_axis_name="s",
                                  num_cores=NC, num_subcores=NS)
    return pl.kernel(
        _pass_a_body,
        out_type=[
            jax.ShapeDtypeStruct((E * H,), jnp.float32),
            jax.ShapeDtypeStruct((E * H,), jnp.float32),
            jax.ShapeDtypeStruct((DN,), jnp.float32),
            jax.ShapeDtypeStruct((DN,), jnp.float32),
            jax.ShapeDtypeStruct((DN,), jnp.float32),
            jax.ShapeDtypeStruct((DN,), jnp.float32),
        ],
        mesh=mesh,
        scratch_types=[
            pltpu.VMEM((CA,), jnp.int32),
            pltpu.VMEM((CA,), jnp.int32),
            pltpu.VMEM((CA * H,), jnp.float32),
            pltpu.VMEM((CA * H,), jnp.int32),
            pltpu.VMEM((CA,), jnp.int32),
            pltpu.VMEM((CA,), jnp.int32),
            pltpu.VMEM((CA * H,), jnp.float32),
            pltpu.VMEM((CA * H,), jnp.int32),
            pltpu.VMEM((N4,), jnp.float32),
            pltpu.VMEM((N4,), jnp.float32),
            pltpu.VMEM_SHARED((DN,), jnp.float32),
            pltpu.SemaphoreType.DMA,
            pltpu.SemaphoreType.DMA,
            pltpu.SemaphoreType.DMA,
            pltpu.SemaphoreType.DMA,
            pltpu.SemaphoreType.DMA,
            pltpu.SemaphoreType.DMA,
        ],
        compiler_params=_params,
    )


def _pass_a2_body(dst0, dst1, ex0, ex1, dnA0, dnB0, dnA1, dnB1,
                  al0, al1,
                  dst_a, exv_a, av_a, dst_b, exv_b, av_b,
                  b1, b2, dn_t, lsem_a, lsem_b, stsem_a, stsem_b):
    core = lax.axis_index("c")
    sid = lax.axis_index("s")
    wid = sid * NC + core
    i16 = jnp.arange(16, dtype=jnp.int32)

    bufs_a = (dst_a, exv_a, av_a, lsem_a, stsem_a)
    bufs_b = (dst_b, exv_b, av_b, lsem_b, stsem_b)
    NCHA = EPW // CA

    def layer(dst_h, ex_h, dnA, dnB, al_h):
        def sb(q, _):
            c1 = pltpu.async_copy(dnA.at[pl.ds(q * 4000, 4000)], b1, lsem_a)
            c2 = pltpu.async_copy(dnB.at[pl.ds(q * 4000, 4000)], b2, lsem_b)
            c1.wait()
            c2.wait()

            def ib(j, _):
                dn_t[pl.ds(q * 4000 + j * 16, 16)] = 1.0 / (
                    b1[pl.ds(j * 16, 16)] + b2[pl.ds(j * 16, 16)])
                return 0

            lax.fori_loop(0, 250, ib, 0)
            return 0

        lax.fori_loop(0, N4 // 4000, sb, 0)

        lbase = wid * EPW

        def start_idx(t, dv, exv, sem):
            base = lbase + t * CA
            pltpu.async_copy(dst_h.at[pl.ds(base, CA)], dv, sem)
            pltpu.async_copy(ex_h.at[pl.ds(base * 4, CA * 4)], exv, sem)

        def wait_idx(dv, exv, sem):
            pltpu.make_async_copy(dst_h.at[pl.ds(0, CA)], dv, sem).wait()
            pltpu.make_async_copy(ex_h.at[pl.ds(0, CA * 4)], exv, sem).wait()

        def step(t, cur, nxt):
            dvc, exc, avc, lsc, stc = cur
            dvn, exn, avn, lsn, stn = nxt

            @pl.when(t > 0)
            def _():
                base1 = lbase + (t - 1) * CA
                pltpu.make_async_copy(
                    avn, al_h.at[pl.ds(base1 * 4, CA * 4)], stn).wait()

            @pl.when(t + 1 < NCHA)
            def _():
                start_idx(t + 1, dvn, exn, lsn)

            wait_idx(dvc, exc, lsc)

            def ab(j, _):
                p = j * 16 + i16
                k = p >> 2
                h = p & 3
                dv = plsc.load_gather(dvc, [k])
                dn = plsc.load_gather(dn_t, [dv * 4 + h])
                avc[pl.ds(j * 16, 16)] = exc[pl.ds(j * 16, 16)] * dn
                return 0

            lax.fori_loop(0, CA * H // 16, ab, 0)
            base = lbase + t * CA
            pltpu.async_copy(avc, al_h.at[pl.ds(base * 4, CA * 4)], stc)

        start_idx(0, dst_a, exv_a, lsem_a)

        def pair(i, _):
            step(2 * i, bufs_a, bufs_b)
            step(2 * i + 1, bufs_b, bufs_a)
            return 0

        lax.fori_loop(0, NCHA // 2, pair, 0)
        base9 = lbase + (NCHA - 1) * CA
        pltpu.make_async_copy(av_b, al_h.at[pl.ds(base9 * 4, CA * 4)],
                              stsem_b).wait()

    layer(dst0, ex0, dnA0, dnB0, al0)
    layer(dst1, ex1, dnA1, dnB1, al1)


@functools.lru_cache(maxsize=None)
def _pass_a2():
    mesh = plsc.VectorSubcoreMesh(core_axis_name="c", subcore_axis_name="s",
                                  num_cores=NC, num_subcores=NS)
    return pl.kernel(
        _pass_a2_body,
        out_type=[
            jax.ShapeDtypeStruct((E * H,), jnp.float32),
            jax.ShapeDtypeStruct((E * H,), jnp.float32),
        ],
        mesh=mesh,
        scratch_types=[
            pltpu.VMEM((CA,), jnp.int32),
            pltpu.VMEM((CA * H,), jnp.float32),
            pltpu.VMEM((CA * H,), jnp.float32),
            pltpu.VMEM((CA,), jnp.int32),
            pltpu.VMEM((CA * H,), jnp.float32),
            pltpu.VMEM((CA * H,), jnp.float32),
            pltpu.VMEM((4000,), jnp.float32),
            pltpu.VMEM((4000,), jnp.float32),
            pltpu.VMEM((N4,), jnp.float32),
            pltpu.SemaphoreType.DMA,
            pltpu.SemaphoreType.DMA,
            pltpu.SemaphoreType.DMA,
            pltpu.SemaphoreType.DMA,
        ],
        compiler_params=_params,
    )


def _pass_b_body(src0, dst0, src1, dst1, al0, al1, feat0, feat1,
                 rstA0, rstB0, rstA1, rstB1,
                 sv0, dv0, av0, fb0, sv1, dv1, av1, fb1, sv2, dv2, av2, fb2,
                 srcE, dstE, avE, rst_sh,
                 ls0, ls1, ls2, fs0, fs1, fs2, ss0, ss1, ss2, rs0, rs1, rs2):
    core = lax.axis_index("c")
    sid = lax.axis_index("s")
    wid = sid * NC + core
    z16 = jnp.zeros((16,), jnp.float32)

    b0 = (sv0, dv0, av0, fb0, ls0, fs0, ss0, rs0)
    b1 = (sv1, dv1, av1, fb1, ls1, fs1, ss1, rs1)
    b2 = (sv2, dv2, av2, fb2, ls2, fs2, ss2, rs2)

    def layer(src_h, dst_h, al_h, feat_h, rstA, rstB):
        def zrow(k, _):
            for g in range(8):
                fb0[k, pl.ds(g * 16, 16)] = z16
            return 0

        lax.fori_loop(0, CB, zrow, 0)
        for j in range(6):
            pltpu.sync_copy(fb0, rst_sh.at[pl.ds(sid * RSTRIPE + j * CB, CB)])
        pltpu.sync_copy(fb0.at[pl.ds(0, RSTRIPE - 6 * CB)],
                        rst_sh.at[pl.ds(sid * RSTRIPE + 6 * CB,
                                        RSTRIPE - 6 * CB)])
        plsc.subcore_barrier()

        lbase = wid * EPW

        def compute(fb, avv, n):
            def eb(q, _):
                for u in range(2):
                    k = q * 2 + u
                    for h in range(H):
                        s = plsc.load_gather(avv, [jnp.full((16,), k * 4 + h,
                                                            jnp.int32)])
                        for g in range(2):
                            c0 = h * D + g * 16
                            fb[k, pl.ds(c0, 16)] = fb[k, pl.ds(c0, 16)] * s
                return 0

            lax.fori_loop(0, n // 2, eb, 0)

        def step(t, cur, nxt, nx2):
            svc, dvc, avc, fbc, lsc, fsc, ssc, rsc = cur
            svn, dvn, avn, fbn, lsn, fsn, ssn, rsn = nxt
            sv2_, dv2_, av2_, fb2_, ls2_, fs2_, ss2_, rs2_ = nx2

            @pl.when(t >= 2)
            def _():
                pltpu.make_async_copy(fbn, rst_sh.at[dvn], ssn).wait()

            @pl.when(t + 1 < NCHB)
            def _():
                pltpu.make_async_copy(src_h.at[pl.ds(0, CB)], svn, rsn).wait()
                pltpu.async_copy(feat_h.at[svn], fbn, fsn)
                base1 = lbase + (t + 1) * CB
                pltpu.async_copy(dst_h.at[pl.ds(base1, CB)], dvn, lsn)
                pltpu.async_copy(al_h.at[pl.ds(base1 * 4, CB * 4)], avn, lsn)

            pltpu.make_async_copy(feat_h.at[svc], fbc, fsc).wait()

            @pl.when(t + 2 < NCHB)
            def _():
                base2 = lbase + (t + 2) * CB
                pltpu.async_copy(src_h.at[pl.ds(base2, CB)], sv2_, rs2_)

            pltpu.make_async_copy(dst_h.at[pl.ds(0, CB)], dvc, lsc).wait()
            pltpu.make_async_copy(al_h.at[pl.ds(0, CB * 4)], avc, lsc).wait()
            compute(fbc, avc, CB)
            pltpu.async_copy(fbc, rst_sh.at[dvc], ssc, add=True)

        pltpu.async_copy(src_h.at[pl.ds(lbase, CB)], sv0, rs0)
        pltpu.async_copy(dst_h.at[pl.ds(lbase, CB)], dv0, ls0)
        pltpu.async_copy(al_h.at[pl.ds(lbase * 4, CB * 4)], av0, ls0)
        pltpu.make_async_copy(src_h.at[pl.ds(0, CB)], sv0, rs0).wait()
        pltpu.async_copy(feat_h.at[sv0], fb0, fs0)
        pltpu.async_copy(src_h.at[pl.ds(lbase + CB, CB)], sv1, rs1)

        def trip(i, _):
            step(3 * i, b0, b1, b2)
            step(3 * i + 1, b1, b2, b0)
            step(3 * i + 2, b2, b0, b1)
            return 0

        lax.fori_loop(0, NCHB // 3, trip, 0)
        pltpu.make_async_copy(fb1, rst_sh.at[dv1], ss1).wait()
        pltpu.make_async_copy(fb2, rst_sh.at[dv2], ss2).wait()

        ebase = lbase + NCHB * CB
        pltpu.sync_copy(src_h.at[pl.ds(ebase, EPI)], srcE)
        pltpu.sync_copy(dst_h.at[pl.ds(ebase, EPI)], dstE)
        pltpu.sync_copy(al_h.at[pl.ds(ebase * 4, EPI * 4)], avE)
        pltpu.async_copy(feat_h.at[srcE], fb0.at[pl.ds(0, EPI)], fs0).wait()
        compute(fb0, avE, EPI)
        pltpu.sync_copy(fb0.at[pl.ds(0, EPI)], rst_sh.at[dstE], add=True)

        plsc.subcore_barrier()

        @pl.when(core == 0)
        def _():
            pltpu.sync_copy(rst_sh.at[pl.ds(sid * RSTRIPE, RSTRIPE)],
                            rstA.at[pl.ds(sid * RSTRIPE, RSTRIPE)])

        @pl.when(core == 1)
        def _():
            pltpu.sync_copy(rst_sh.at[pl.ds(sid * RSTRIPE, RSTRIPE)],
                            rstB.at[pl.ds(sid * RSTRIPE, RSTRIPE)])

        plsc.subcore_barrier()

    layer(src0, dst0, al0, feat0, rstA0, rstB0)
    layer(src1, dst1, al1, feat1, rstA1, rstB1)


@functools.lru_cache(maxsize=None)
def _pass_b():
    mesh = plsc.VectorSubcoreMesh(core_axis_name="c", subcore_axis_name="s",
                                  num_cores=NC, num_subcores=NS)
    dma = pltpu.SemaphoreType.DMA
    return pl.kernel(
        _pass_b_body,
        out_type=[
            jax.ShapeDtypeStruct((NR, DIM), jnp.float32),
            jax.ShapeDtypeStruct((NR, DIM), jnp.float32),
            jax.ShapeDtypeStruct((NR, DIM), jnp.float32),
            jax.ShapeDtypeStruct((NR, DIM), jnp.float32),
        ],
        mesh=mesh,
        scratch_types=(
            [pltpu.VMEM((CB,), jnp.int32),
             pltpu.VMEM((CB,), jnp.int32),
             pltpu.VMEM((CB * H,), jnp.float32),
             pltpu.VMEM((CB, DIM), jnp.float32)] * 3
            + [pltpu.VMEM((EPI,), jnp.int32),
               pltpu.VMEM((EPI,), jnp.int32),
               pltpu.VMEM((EPI * H,), jnp.float32),
               pltpu.VMEM_SHARED((NR, DIM), jnp.float32)]
            + [dma] * 12
        ),
        compiler_params=_params,
    )


def _expand_att(a):
    rows = jnp.arange(DIM)
    m = (rows[:, None] // D) == jnp.arange(H)[None, :]
    return jnp.where(m, a.reshape(-1)[:, None], 0.0).astype(jnp.float32)


def kernel(x0, x1, edge_index0, edge_index1, W0, al0, ar0, W1, al1, ar1):
    feat0, el0, er0 = _tc_pre(x0, W0, _expand_att(al0), _expand_att(ar0))
    feat1, el1, er1 = _tc_pre(x1, W1, _expand_att(al1), _expand_att(ar1))

    src0, dst0 = edge_index0[0], edge_index0[1]
    src1, dst1 = edge_index1[0], edge_index1[1]

    ex0, ex1, dnA0, dnB0, dnA1, dnB1 = _pass_a()(
        src0, dst0, src1, dst1,
        el0.reshape(-1), er0.reshape(-1), el1.reshape(-1), er1.reshape(-1))

    al0_, al1_ = _pass_a2()(
        dst0, dst1, ex0, ex1, dnA0, dnB0, dnA1, dnB1)

    rstA0, rstB0, rstA1, rstB1 = _pass_b()(
        src0, dst0, src1, dst1, al0_, al1_, feat0, feat1)

    h0 = _tc_post(rstA0, rstB0, x0)
    h1 = _tc_post(rstA1, rstB1, x1)

    return (h0, h1,
            al0_.reshape(E, H, 1), al1_.reshape(E, H, 1))


# speedup vs baseline: 1.2032x; 1.0149x over previous
import functools

import jax
import jax.numpy as jnp
from jax import lax
from jax.experimental import pallas as pl
from jax.experimental.pallas import tpu as pltpu
from jax.experimental.pallas import tpu_sc as plsc

N = 10000
E = 320000
H = 4
D = 32
DIM = 128

NC = 2
NS = 16
NW = NC * NS
EPW = E // NW
N4 = N * H
DN = 40960
DSTRIPE = DN // NS
NR = 10112
RSTRIPE = NR // NS

CA = 1000
CB = 104
NCHB = 96
EPI = EPW - NCHB * CB

_params = pltpu.CompilerParams(needs_layout_passes=False)


def _pre_body(x0_ref, w0_ref, alm0_ref, arm0_ref,
              x1_ref, w1_ref, alm1_ref, arm1_ref,
              feat0_ref, el0_ref, er0_ref, feat1_ref, el1_ref, er1_ref):
    f0 = jnp.dot(x0_ref[...], w0_ref[...], preferred_element_type=jnp.float32)
    feat0_ref[...] = f0
    el0_ref[...] = jnp.dot(f0, alm0_ref[...], preferred_element_type=jnp.float32)
    er0_ref[...] = jnp.dot(f0, arm0_ref[...], preferred_element_type=jnp.float32)
    f1 = jnp.dot(x1_ref[...], w1_ref[...], preferred_element_type=jnp.float32)
    feat1_ref[...] = f1
    el1_ref[...] = jnp.dot(f1, alm1_ref[...], preferred_element_type=jnp.float32)
    er1_ref[...] = jnp.dot(f1, arm1_ref[...], preferred_element_type=jnp.float32)


def _tc_pre(x0, W0, alm0, arm0, x1, W1, alm1, arm1):
    R = 1000
    row = pl.BlockSpec((R, DIM), lambda i: (i, 0))
    small = pl.BlockSpec((R, H), lambda i: (i, 0))
    full = pl.BlockSpec((DIM, DIM), lambda i: (0, 0))
    att = pl.BlockSpec((DIM, H), lambda i: (0, 0))
    return pl.pallas_call(
        _pre_body,
        grid=(N // R,),
        in_specs=[row, full, att, att, row, full, att, att],
        out_specs=[row, small, small, row, small, small],
        out_shape=[
            jax.ShapeDtypeStruct((N, DIM), jnp.float32),
            jax.ShapeDtypeStruct((N, H), jnp.float32),
            jax.ShapeDtypeStruct((N, H), jnp.float32),
            jax.ShapeDtypeStruct((N, DIM), jnp.float32),
            jax.ShapeDtypeStruct((N, H), jnp.float32),
            jax.ShapeDtypeStruct((N, H), jnp.float32),
        ],
    )(x0, W0, alm0, arm0, x1, W1, alm1, arm1)


def _post_body(pa0_ref, pb0_ref, x0_ref, pa1_ref, pb1_ref, x1_ref,
               o0_ref, o1_ref):
    r0 = pa0_ref[...] + pb0_ref[...] + x0_ref[...]
    o0_ref[...] = jnp.where(r0 > 0.0, r0, jnp.exp(r0) - 1.0)
    r1 = pa1_ref[...] + pb1_ref[...] + x1_ref[...]
    o1_ref[...] = jnp.where(r1 > 0.0, r1, jnp.exp(r1) - 1.0)


def _tc_post(pa0, pb0, x0, pa1, pb1, x1):
    R = 1000
    row = pl.BlockSpec((R, DIM), lambda i: (i, 0))
    return pl.pallas_call(
        _post_body,
        grid=(N // R,),
        in_specs=[row] * 6,
        out_specs=[row, row],
        out_shape=[jax.ShapeDtypeStruct((N, DIM), jnp.float32),
                   jax.ShapeDtypeStruct((N, DIM), jnp.float32)],
    )(pa0, pb0, x0, pa1, pb1, x1)


def _pass_a_body(src0, dst0, src1, dst1, el0, er0, el1, er1,
                 ex0, ex1, dnA0, dnB0, dnA1, dnB1,
                 src_a, dst_a, exv_a, idx4_a, src_b, dst_b, exv_b, idx4_b,
                 el_t, er_t, dn_sh,
                 lsem_a, lsem_b, stsem_a, stsem_b, scsem_a, scsem_b):
    core = lax.axis_index("c")
    sid = lax.axis_index("s")
    wid = sid * NC + core
    i16 = jnp.arange(16, dtype=jnp.int32)
    z16 = jnp.zeros((16,), jnp.float32)

    bufs_a = (src_a, dst_a, exv_a, idx4_a, lsem_a, stsem_a, scsem_a)
    bufs_b = (src_b, dst_b, exv_b, idx4_b, lsem_b, stsem_b, scsem_b)
    NCHA = EPW // CA

    def layer(src_h, dst_h, el_h, er_h, ex_h, dnA, dnB):
        c1 = pltpu.async_copy(el_h, el_t, lsem_a)
        c2 = pltpu.async_copy(er_h, er_t, lsem_b)

        def zb(j, _):
            exv_a[pl.ds(j * 16, 16)] = z16
            return 0

        lax.fori_loop(0, DSTRIPE // 16, zb, 0)
        pltpu.sync_copy(exv_a.at[pl.ds(0, DSTRIPE)],
                        dn_sh.at[pl.ds(sid * DSTRIPE, DSTRIPE)])
        c1.wait()
        c2.wait()
        plsc.subcore_barrier()

        lbase = wid * EPW

        def start_idx(t, sv, dv, sem):
            base = lbase + t * CA
            pltpu.async_copy(src_h.at[pl.ds(base, CA)], sv, sem)
            pltpu.async_copy(dst_h.at[pl.ds(base, CA)], dv, sem)

        def wait_idx(sv, dv, sem):
            pltpu.make_async_copy(src_h.at[pl.ds(0, CA)], sv, sem).wait()
            pltpu.make_async_copy(dst_h.at[pl.ds(0, CA)], dv, sem).wait()

        def step(t, cur, nxt):
            svc, dvc, exc, idc, lsc, stc, scc = cur
            svn, dvn, exn, idn, lsn, stn, scn = nxt

            @pl.when(t > 0)
            def _():
                base1 = lbase + (t - 1) * CA
                pltpu.make_async_copy(
                    exn, ex_h.at[pl.ds(base1 * 4, CA * 4)], stn).wait()
                pltpu.make_async_copy(exn, dn_sh.at[idn], scn).wait()

            @pl.when(t + 1 < NCHA)
            def _():
                start_idx(t + 1, svn, dvn, lsn)

            wait_idx(svc, dvc, lsc)

            def eb(q, _):
                for u in range(2):
                    j = q * 2 + u
                    p = j * 16 + i16
                    k = p >> 2
                    h = p & 3
                    sv = plsc.load_gather(svc, [k])
                    dv = plsc.load_gather(dvc, [k])
                    e = (plsc.load_gather(el_t, [sv * 4 + h])
                         + plsc.load_gather(er_t, [dv * 4 + h]))
                    e = jnp.where(e >= 0.0, e, 0.2 * e)
                    exc[pl.ds(j * 16, 16)] = jnp.exp(e)
                    idc[pl.ds(j * 16, 16)] = dv * 4 + h
                return 0

            lax.fori_loop(0, CA * H // 32, eb, 0)
            base = lbase + t * CA
            pltpu.async_copy(exc, ex_h.at[pl.ds(base * 4, CA * 4)], stc)
            pltpu.async_copy(exc, dn_sh.at[idc], scc, add=True)

        start_idx(0, src_a, dst_a, lsem_a)

        def pair(i, _):
            step(2 * i, bufs_a, bufs_b)
            step(2 * i + 1, bufs_b, bufs_a)
            return 0

        lax.fori_loop(0, NCHA // 2, pair, 0)
        base9 = lbase + (NCHA - 1) * CA
        pltpu.make_async_copy(exv_b, ex_h.at[pl.ds(base9 * 4, CA * 4)],
                              stsem_b).wait()
        pltpu.make_async_copy(exv_b, dn_sh.at[idx4_b], scsem_b).wait()
        plsc.subcore_barrier()

        @pl.when(core == 0)
        def _():
            pltpu.sync_copy(dn_sh.at[pl.ds(sid * DSTRIPE, DSTRIPE)],
                            dnA.at[pl.ds(sid * DSTRIPE, DSTRIPE)])

        @pl.when(core == 1)
        def _():
            pltpu.sync_copy(dn_sh.at[pl.ds(sid * DSTRIPE, DSTRIPE)],
                            dnB.at[pl.ds(sid * DSTRIPE, DSTRIPE)])

        plsc.subcore_barrier()

    layer(src0, dst0, el0, er0, ex0, dnA0, dnB0)
    layer(src1, dst1, el1, er1, ex1, dnA1, dnB1)


@functools.lru_cache(maxsize=None)
def _pass_a():
    mesh = plsc.VectorSubcoreMesh(core_axis_name="c", subcore_axis_name="s",
                                  num_cores=NC, num_subcores=NS)
    return pl.kernel(
        _pass_a_body,
        out_type=[
            jax.ShapeDtypeStruct((E * H,), jnp.float32),
            jax.ShapeDtypeStruct((E * H,), jnp.float32),
            jax.ShapeDtypeStruct((DN,), jnp.float32),
            jax.ShapeDtypeStruct((DN,), jnp.float32),
            jax.ShapeDtypeStruct((DN,), jnp.float32),
            jax.ShapeDtypeStruct((DN,), jnp.float32),
        ],
        mesh=mesh,
        scratch_types=[
            pltpu.VMEM((CA,), jnp.int32),
            pltpu.VMEM((CA,), jnp.int32),
            pltpu.VMEM((CA * H,), jnp.float32),
            pltpu.VMEM((CA * H,), jnp.int32),
            pltpu.VMEM((CA,), jnp.int32),
            pltpu.VMEM((CA,), jnp.int32),
            pltpu.VMEM((CA * H,), jnp.float32),
            pltpu.VMEM((CA * H,), jnp.int32),
            pltpu.VMEM((N4,), jnp.float32),
            pltpu.VMEM((N4,), jnp.float32),
            pltpu.VMEM_SHARED((DN,), jnp.float32),
            pltpu.SemaphoreType.DMA,
            pltpu.SemaphoreType.DMA,
            pltpu.SemaphoreType.DMA,
            pltpu.SemaphoreType.DMA,
            pltpu.SemaphoreType.DMA,
            pltpu.SemaphoreType.DMA,
        ],
        compiler_params=_params,
    )


def _pass_a2_body(dst0, dst1, ex0, ex1, dnA0, dnB0, dnA1, dnB1,
                  al0, al1,
                  dst_a, exv_a, av_a, dst_b, exv_b, av_b,
                  b1, b2, dn_t, lsem_a, lsem_b, stsem_a, stsem_b):
    core = lax.axis_index("c")
    sid = lax.axis_index("s")
    wid = sid * NC + core
    i16 = jnp.arange(16, dtype=jnp.int32)

    bufs_a = (dst_a, exv_a, av_a, lsem_a, stsem_a)
    bufs_b = (dst_b, exv_b, av_b, lsem_b, stsem_b)
    NCHA = EPW // CA

    def layer(dst_h, ex_h, dnA, dnB, al_h):
        def sb(q, _):
            c1 = pltpu.async_copy(dnA.at[pl.ds(q * 4000, 4000)], b1, lsem_a)
            c2 = pltpu.async_copy(dnB.at[pl.ds(q * 4000, 4000)], b2, lsem_b)
            c1.wait()
            c2.wait()

            def ib(j, _):
                dn_t[pl.ds(q * 4000 + j * 16, 16)] = 1.0 / (
                    b1[pl.ds(j * 16, 16)] + b2[pl.ds(j * 16, 16)])
                return 0

            lax.fori_loop(0, 250, ib, 0)
            return 0

        lax.fori_loop(0, N4 // 4000, sb, 0)

        lbase = wid * EPW

        def start_idx(t, dv, exv, sem):
            base = lbase + t * CA
            pltpu.async_copy(dst_h.at[pl.ds(base, CA)], dv, sem)
            pltpu.async_copy(ex_h.at[pl.ds(base * 4, CA * 4)], exv, sem)

        def wait_idx(dv, exv, sem):
            pltpu.make_async_copy(dst_h.at[pl.ds(0, CA)], dv, sem).wait()
            pltpu.make_async_copy(ex_h.at[pl.ds(0, CA * 4)], exv, sem).wait()

        def step(t, cur, nxt):
            dvc, exc, avc, lsc, stc = cur
            dvn, exn, avn, lsn, stn = nxt

            @pl.when(t > 0)
            def _():
                base1 = lbase + (t - 1) * CA
                pltpu.make_async_copy(
                    avn, al_h.at[pl.ds(base1 * 4, CA * 4)], stn).wait()

            @pl.when(t + 1 < NCHA)
            def _():
                start_idx(t + 1, dvn, exn, lsn)

            wait_idx(dvc, exc, lsc)

            def ab(q, _):
                for u in range(2):
                    j = q * 2 + u
                    p = j * 16 + i16
                    k = p >> 2
                    h = p & 3
                    dv = plsc.load_gather(dvc, [k])
                    dn = plsc.load_gather(dn_t, [dv * 4 + h])
                    avc[pl.ds(j * 16, 16)] = exc[pl.ds(j * 16, 16)] * dn
                return 0

            lax.fori_loop(0, CA * H // 32, ab, 0)
            base = lbase + t * CA
            pltpu.async_copy(avc, al_h.at[pl.ds(base * 4, CA * 4)], stc)

        start_idx(0, dst_a, exv_a, lsem_a)

        def pair(i, _):
            step(2 * i, bufs_a, bufs_b)
            step(2 * i + 1, bufs_b, bufs_a)
            return 0

        lax.fori_loop(0, NCHA // 2, pair, 0)
        base9 = lbase + (NCHA - 1) * CA
        pltpu.make_async_copy(av_b, al_h.at[pl.ds(base9 * 4, CA * 4)],
                              stsem_b).wait()

    layer(dst0, ex0, dnA0, dnB0, al0)
    layer(dst1, ex1, dnA1, dnB1, al1)


@functools.lru_cache(maxsize=None)
def _pass_a2():
    mesh = plsc.VectorSubcoreMesh(core_axis_name="c", subcore_axis_name="s",
                                  num_cores=NC, num_subcores=NS)
    return pl.kernel(
        _pass_a2_body,
        out_type=[
            jax.ShapeDtypeStruct((E * H,), jnp.float32),
            jax.ShapeDtypeStruct((E * H,), jnp.float32),
        ],
        mesh=mesh,
        scratch_types=[
            pltpu.VMEM((CA,), jnp.int32),
            pltpu.VMEM((CA * H,), jnp.float32),
            pltpu.VMEM((CA * H,), jnp.float32),
            pltpu.VMEM((CA,), jnp.int32),
            pltpu.VMEM((CA * H,), jnp.float32),
            pltpu.VMEM((CA * H,), jnp.float32),
            pltpu.VMEM((4000,), jnp.float32),
            pltpu.VMEM((4000,), jnp.float32),
            pltpu.VMEM((N4,), jnp.float32),
            pltpu.SemaphoreType.DMA,
            pltpu.SemaphoreType.DMA,
            pltpu.SemaphoreType.DMA,
            pltpu.SemaphoreType.DMA,
        ],
        compiler_params=_params,
    )


def _pass_b_body(src0, dst0, src1, dst1, al0, al1, feat0, feat1,
                 rstA0, rstB0, rstA1, rstB1,
                 sv0, dv0, av0, fb0, sv1, dv1, av1, fb1, sv2, dv2, av2, fb2,
                 srcE, dstE, avE, rst_sh,
                 ls0, ls1, ls2, fs0, fs1, fs2, ss0, ss1, ss2, rs0, rs1, rs2):
    core = lax.axis_index("c")
    sid = lax.axis_index("s")
    wid = sid * NC + core
    z16 = jnp.zeros((16,), jnp.float32)

    b0 = (sv0, dv0, av0, fb0, ls0, fs0, ss0, rs0)
    b1 = (sv1, dv1, av1, fb1, ls1, fs1, ss1, rs1)
    b2 = (sv2, dv2, av2, fb2, ls2, fs2, ss2, rs2)

    def layer(src_h, dst_h, al_h, feat_h, rstA, rstB):
        def zrow(k, _):
            for g in range(8):
                fb0[k, pl.ds(g * 16, 16)] = z16
            return 0

        lax.fori_loop(0, CB, zrow, 0)
        for j in range(6):
            pltpu.sync_copy(fb0, rst_sh.at[pl.ds(sid * RSTRIPE + j * CB, CB)])
        pltpu.sync_copy(fb0.at[pl.ds(0, RSTRIPE - 6 * CB)],
                        rst_sh.at[pl.ds(sid * RSTRIPE + 6 * CB,
                                        RSTRIPE - 6 * CB)])
        plsc.subcore_barrier()

        lbase = wid * EPW

        def compute(fb, avv, n):
            def eb(q, _):
                for u in range(2):
                    k = q * 2 + u
                    for h in range(H):
                        s = plsc.load_gather(avv, [jnp.full((16,), k * 4 + h,
                                                            jnp.int32)])
                        for g in range(2):
                            c0 = h * D + g * 16
                            fb[k, pl.ds(c0, 16)] = fb[k, pl.ds(c0, 16)] * s
                return 0

            lax.fori_loop(0, n // 2, eb, 0)

        def step(t, cur, nxt, nx2):
            svc, dvc, avc, fbc, lsc, fsc, ssc, rsc = cur
            svn, dvn, avn, fbn, lsn, fsn, ssn, rsn = nxt
            sv2_, dv2_, av2_, fb2_, ls2_, fs2_, ss2_, rs2_ = nx2

            @pl.when(t >= 2)
            def _():
                pltpu.make_async_copy(fbn, rst_sh.at[dvn], ssn).wait()

            @pl.when(t + 1 < NCHB)
            def _():
                pltpu.make_async_copy(src_h.at[pl.ds(0, CB)], svn, rsn).wait()
                pltpu.async_copy(feat_h.at[svn], fbn, fsn)
                base1 = lbase + (t + 1) * CB
                pltpu.async_copy(dst_h.at[pl.ds(base1, CB)], dvn, lsn)
                pltpu.async_copy(al_h.at[pl.ds(base1 * 4, CB * 4)], avn, lsn)

            pltpu.make_async_copy(feat_h.at[svc], fbc, fsc).wait()

            @pl.when(t + 2 < NCHB)
            def _():
                base2 = lbase + (t + 2) * CB
                pltpu.async_copy(src_h.at[pl.ds(base2, CB)], sv2_, rs2_)

            pltpu.make_async_copy(dst_h.at[pl.ds(0, CB)], dvc, lsc).wait()
            pltpu.make_async_copy(al_h.at[pl.ds(0, CB * 4)], avc, lsc).wait()
            compute(fbc, avc, CB)
            pltpu.async_copy(fbc, rst_sh.at[dvc], ssc, add=True)

        pltpu.async_copy(src_h.at[pl.ds(lbase, CB)], sv0, rs0)
        pltpu.async_copy(dst_h.at[pl.ds(lbase, CB)], dv0, ls0)
        pltpu.async_copy(al_h.at[pl.ds(lbase * 4, CB * 4)], av0, ls0)
        pltpu.make_async_copy(src_h.at[pl.ds(0, CB)], sv0, rs0).wait()
        pltpu.async_copy(feat_h.at[sv0], fb0, fs0)
        pltpu.async_copy(src_h.at[pl.ds(lbase + CB, CB)], sv1, rs1)

        def trip(i, _):
            step(3 * i, b0, b1, b2)
            step(3 * i + 1, b1, b2, b0)
            step(3 * i + 2, b2, b0, b1)
            return 0

        lax.fori_loop(0, NCHB // 3, trip, 0)
        pltpu.make_async_copy(fb1, rst_sh.at[dv1], ss1).wait()
        pltpu.make_async_copy(fb2, rst_sh.at[dv2], ss2).wait()

        ebase = lbase + NCHB * CB
        pltpu.sync_copy(src_h.at[pl.ds(ebase, EPI)], srcE)
        pltpu.sync_copy(dst_h.at[pl.ds(ebase, EPI)], dstE)
        pltpu.sync_copy(al_h.at[pl.ds(ebase * 4, EPI * 4)], avE)
        pltpu.async_copy(feat_h.at[srcE], fb0.at[pl.ds(0, EPI)], fs0).wait()
        compute(fb0, avE, EPI)
        pltpu.sync_copy(fb0.at[pl.ds(0, EPI)], rst_sh.at[dstE], add=True)

        plsc.subcore_barrier()

        @pl.when(core == 0)
        def _():
            pltpu.sync_copy(rst_sh.at[pl.ds(sid * RSTRIPE, RSTRIPE)],
                            rstA.at[pl.ds(sid * RSTRIPE, RSTRIPE)])

        @pl.when(core == 1)
        def _():
            pltpu.sync_copy(rst_sh.at[pl.ds(sid * RSTRIPE, RSTRIPE)],
                            rstB.at[pl.ds(sid * RSTRIPE, RSTRIPE)])

        plsc.subcore_barrier()

    layer(src0, dst0, al0, feat0, rstA0, rstB0)
    layer(src1, dst1, al1, feat1, rstA1, rstB1)


@functools.lru_cache(maxsize=None)
def _pass_b():
    mesh = plsc.VectorSubcoreMesh(core_axis_name="c", subcore_axis_name="s",
                                  num_cores=NC, num_subcores=NS)
    dma = pltpu.SemaphoreType.DMA
    return pl.kernel(
        _pass_b_body,
        out_type=[
            jax.ShapeDtypeStruct((NR, DIM), jnp.float32),
            jax.ShapeDtypeStruct((NR, DIM), jnp.float32),
            jax.ShapeDtypeStruct((NR, DIM), jnp.float32),
            jax.ShapeDtypeStruct((NR, DIM), jnp.float32),
        ],
        mesh=mesh,
        scratch_types=(
            [pltpu.VMEM((CB,), jnp.int32),
             pltpu.VMEM((CB,), jnp.int32),
             pltpu.VMEM((CB * H,), jnp.float32),
             pltpu.VMEM((CB, DIM), jnp.float32)] * 3
            + [pltpu.VMEM((EPI,), jnp.int32),
               pltpu.VMEM((EPI,), jnp.int32),
               pltpu.VMEM((EPI * H,), jnp.float32),
               pltpu.VMEM_SHARED((NR, DIM), jnp.float32)]
            + [dma] * 12
        ),
        compiler_params=_params,
    )


def _expand_att(a):
    rows = jnp.arange(DIM)
    m = (rows[:, None] // D) == jnp.arange(H)[None, :]
    return jnp.where(m, a.reshape(-1)[:, None], 0.0).astype(jnp.float32)


def kernel(x0, x1, edge_index0, edge_index1, W0, al0, ar0, W1, al1, ar1):
    feat0, el0, er0, feat1, el1, er1 = _tc_pre(
        x0, W0, _expand_att(al0), _expand_att(ar0),
        x1, W1, _expand_att(al1), _expand_att(ar1))

    src0, dst0 = edge_index0[0], edge_index0[1]
    src1, dst1 = edge_index1[0], edge_index1[1]

    ex0, ex1, dnA0, dnB0, dnA1, dnB1 = _pass_a()(
        src0, dst0, src1, dst1,
        el0.reshape(-1), er0.reshape(-1), el1.reshape(-1), er1.reshape(-1))

    al0_, al1_ = _pass_a2()(
        dst0, dst1, ex0, ex1, dnA0, dnB0, dnA1, dnB1)

    rstA0, rstB0, rstA1, rstB1 = _pass_b()(
        src0, dst0, src1, dst1, al0_, al1_, feat0, feat1)

    h0, h1 = _tc_post(rstA0, rstB0, x0, rstA1, rstB1, x1)

    return (h0, h1,
            al0_.reshape(E, H, 1), al1_.reshape(E, H, 1))
